# baseline (speedup 1.0000x reference)
import functools
import jax
import jax.numpy as jnp
from jax import lax
from jax.experimental import pallas as pl
from jax.experimental.pallas import tpu as pltpu
from jax.experimental.pallas import tpu_sc as plsc

B, D, H, W = 2, 64, 128, 128
NVOX = B * D * H * W
NREG = 4
NC, NS, L = 2, 16, 16
NW = NC * NS
CHUNK = NVOX // NW
TILE = 8192
NT = CHUNK // TILE
NACC = 3 * NREG


def _sc_body(x_hbm, ml_hbm, rm_hbm, out_hbm, xv, mlv, rmv, acc, sem):
    wid = lax.axis_index("s") * NC + lax.axis_index("c")
    base = wid * CHUNK
    for j in range(NACC):
        acc[pl.ds(j * L, L)] = jnp.zeros((L,), jnp.float32)
    lanes = lax.iota(jnp.int32, L)

    def tile_body(t, carry):
        off = base + t * TILE
        pltpu.sync_copy(x_hbm.at[pl.ds(off, TILE)], xv)
        pltpu.sync_copy(ml_hbm.at[pl.ds(off, TILE)], mlv)
        pltpu.sync_copy(rm_hbm.at[pl.ds(off, TILE)], rmv)

        def vec_body(i, c):
            p = i * L
            xr = xv[pl.ds(p, L)]
            y = mlv[pl.ds(p, L)].astype(jnp.float32)
            idx = rmv[pl.ds(p, L)] - 1
            s = 1.0 / (1.0 + jnp.exp(-xr))
            flat = idx * L + lanes
            plsc.addupdate_scatter(acc, [flat], s * y)
            plsc.addupdate_scatter(acc, [flat + NREG * L], s)
            plsc.addupdate_scatter(acc, [flat + 2 * NREG * L], y)
            return c

        lax.fori_loop(0, TILE // L, vec_body, 0, unroll=4)
        return carry

    lax.fori_loop(0, NT, tile_body, 0)
    pltpu.sync_copy(acc, out_hbm.at[wid])


_sc_kernel = functools.partial(
    pl.kernel,
    out_type=jax.ShapeDtypeStruct((NW, NACC * L), jnp.float32),
    mesh=plsc.VectorSubcoreMesh(core_axis_name="c", subcore_axis_name="s",
                                num_cores=NC, num_subcores=NS),
    compiler_params=pltpu.CompilerParams(needs_layout_passes=False),
    scratch_types=[
        pltpu.VMEM((TILE,), jnp.float32),
        pltpu.VMEM((TILE,), jnp.int32),
        pltpu.VMEM((TILE,), jnp.int32),
        pltpu.VMEM((NACC * L,), jnp.float32),
        pltpu.SemaphoreType.DMA,
    ],
)(_sc_body)


def _combine_body(p_ref, o_ref):
    total = 0.0
    for b in range(B):
        rows = p_ref[b * (NW // B):(b + 1) * (NW // B), :]
        mean_d = 0.0
        for r in range(NREG):
            i_sum = jnp.sum(rows[:, r * L:(r + 1) * L])
            a_sum = jnp.sum(rows[:, (NREG + r) * L:(NREG + r + 1) * L])
            y_sum = jnp.sum(rows[:, (2 * NREG + r) * L:(2 * NREG + r + 1) * L])
            mean_d += 2.0 * i_sum / (a_sum + y_sum)
        total += 1.0 - mean_d / NREG
    o_ref[...] = jnp.full((1, 1), total / B, jnp.float32)


def kernel(x, multi_label, region_map):
    xf = x.reshape(NVOX)
    mlf = multi_label.reshape(NVOX)
    rmf = region_map.reshape(NVOX)
    partials = _sc_kernel(xf, mlf, rmf)
    out = pl.pallas_call(
        _combine_body,
        out_shape=jax.ShapeDtypeStruct((1, 1), jnp.float32),
    )(partials)
    return out.reshape(())


# speedup vs baseline: 3.7204x; 3.7204x over previous
import functools
import jax
import jax.numpy as jnp
from jax import lax
from jax.experimental import pallas as pl
from jax.experimental.pallas import tpu as pltpu
from jax.experimental.pallas import tpu_sc as plsc

B, D, H, W = 2, 64, 128, 128
NVOX = B * D * H * W
NREG = 4
NLBL = 8
NC, NS, L = 2, 16, 16
NW = NC * NS
CHUNK = NVOX // NW
TILE = 16384
NT = CHUNK // TILE
NBANK = 8
NROW = NREG * NLBL
BANKSZ = NROW * L
NACC = 3 * NREG


def _sc_body(x_hbm, ml_hbm, rm_hbm, out_hbm,
             xv0, mlv0, rmv0, xv1, mlv1, rmv1,
             acc_s, acc_c, outv, sem0, sem1):
    wid = lax.axis_index("s") * NC + lax.axis_index("c")
    base = wid * CHUNK
    bufs = ((xv0, mlv0, rmv0, sem0), (xv1, mlv1, rmv1, sem1))

    zero = jnp.zeros((L,), jnp.float32)
    for j in range(NBANK * NROW):
        acc_s[pl.ds(j * L, L)] = zero
        acc_c[pl.ds(j * L, L)] = zero

    laneconst = lax.iota(jnp.int32, L) - NLBL * L
    ones = jnp.full((L,), 1.0, jnp.float32)

    def issue(t):
        xv, mlv, rmv, sem = bufs[t % 2]
        off = base + t * TILE
        return (
            pltpu.make_async_copy(x_hbm.at[pl.ds(off, TILE)], xv, sem),
            pltpu.make_async_copy(ml_hbm.at[pl.ds(off, TILE)], mlv, sem),
            pltpu.make_async_copy(rm_hbm.at[pl.ds(off, TILE)], rmv, sem),
        )

    def start(handles):
        for h in handles:
            h.start()

    pending = issue(0)
    start(pending)
    for t in range(NT):
        if t + 1 < NT:
            nxt = issue(t + 1)
            start(nxt)
        else:
            nxt = None
        for h in pending:
            h.wait()
        xv, mlv, rmv, _ = bufs[t % 2]

        @functools.partial(plsc.parallel_loop, 0, TILE // L, unroll=8)
        def _(i):
            p = i * L
            xr = xv[pl.ds(p, L)]
            ml = mlv[pl.ds(p, L)]
            rm = rmv[pl.ds(p, L)]
            s = 1.0 / (1.0 + jnp.exp(-xr))
            bucket = (((rm << 3) + ml) << 4) + laneconst + ((i & (NBANK - 1)) << 9)
            plsc.addupdate_scatter(acc_s, [bucket], s)
            plsc.addupdate_scatter(acc_c, [bucket], ones)

        pending = nxt

    for r in range(NREG):
        i_vec = zero
        a_vec = zero
        y_vec = zero
        for v in range(NLBL):
            srow = zero
            crow = zero
            for k in range(NBANK):
                off = k * BANKSZ + (r * NLBL + v) * L
                srow = srow + acc_s[pl.ds(off, L)]
                crow = crow + acc_c[pl.ds(off, L)]
            fv = float(v)
            i_vec = i_vec + fv * srow
            a_vec = a_vec + srow
            y_vec = y_vec + fv * crow
        outv[pl.ds(r * L, L)] = i_vec
        outv[pl.ds((NREG + r) * L, L)] = a_vec
        outv[pl.ds((2 * NREG + r) * L, L)] = y_vec
    pltpu.sync_copy(outv, out_hbm.at[wid])


_sc_kernel = functools.partial(
    pl.kernel,
    out_type=jax.ShapeDtypeStruct((NW, NACC * L), jnp.float32),
    mesh=plsc.VectorSubcoreMesh(core_axis_name="c", subcore_axis_name="s",
                                num_cores=NC, num_subcores=NS),
    compiler_params=pltpu.CompilerParams(needs_layout_passes=False),
    scratch_types=[
        pltpu.VMEM((TILE,), jnp.float32),
        pltpu.VMEM((TILE,), jnp.int32),
        pltpu.VMEM((TILE,), jnp.int32),
        pltpu.VMEM((TILE,), jnp.float32),
        pltpu.VMEM((TILE,), jnp.int32),
        pltpu.VMEM((TILE,), jnp.int32),
        pltpu.VMEM((NBANK * BANKSZ,), jnp.float32),
        pltpu.VMEM((NBANK * BANKSZ,), jnp.float32),
        pltpu.VMEM((NACC * L,), jnp.float32),
        pltpu.SemaphoreType.DMA,
        pltpu.SemaphoreType.DMA,
    ],
)(_sc_body)


def _combine_body(p_ref, o_ref):
    total = 0.0
    for b in range(B):
        rows = p_ref[b * (NW // B):(b + 1) * (NW // B), :]
        mean_d = 0.0
        for r in range(NREG):
            i_sum = jnp.sum(rows[:, r * L:(r + 1) * L])
            a_sum = jnp.sum(rows[:, (NREG + r) * L:(NREG + r + 1) * L])
            y_sum = jnp.sum(rows[:, (2 * NREG + r) * L:(2 * NREG + r + 1) * L])
            mean_d += 2.0 * i_sum / (a_sum + y_sum)
        total += 1.0 - mean_d / NREG
    o_ref[...] = jnp.full((1, 1), total / B, jnp.float32)


def kernel(x, multi_label, region_map):
    xf = x.reshape(NVOX)
    mlf = multi_label.reshape(NVOX)
    rmf = region_map.reshape(NVOX)
    partials = _sc_kernel(xf, mlf, rmf)
    out = pl.pallas_call(
        _combine_body,
        out_shape=jax.ShapeDtypeStruct((1, 1), jnp.float32),
    )(partials)
    return out.reshape(())


# speedup vs baseline: 3.9464x; 1.0607x over previous
import functools
import jax
import jax.numpy as jnp
from jax import lax
from jax.experimental import pallas as pl
from jax.experimental.pallas import tpu as pltpu
from jax.experimental.pallas import tpu_sc as plsc

B, D, H, W = 2, 64, 128, 128
NVOX = B * D * H * W
NREG = 4
NLBL = 8
NC, NS, L = 2, 16, 16
NW = NC * NS
CHUNK = NVOX // NW
TILE = 16384
NT = CHUNK // TILE
NBANK = 8
NROW = NREG * NLBL
BANKSZ = NROW * L
NACC = 3 * NREG


def _sc_body(x_hbm, ml_hbm, rm_hbm, out_hbm,
             xv0, mlv0, rmv0, xv1, mlv1, rmv1,
             acc_s, acc_c, outv, sem0, sem1):
    wid = lax.axis_index("s") * NC + lax.axis_index("c")
    base = wid * CHUNK
    bufs = ((xv0, mlv0, rmv0, sem0), (xv1, mlv1, rmv1, sem1))

    zero = jnp.zeros((L,), jnp.float32)

    def zinit(j, c):
        acc_s[pl.ds(j * L, L)] = zero
        acc_c[pl.ds(j * L, L)] = zero
        return c

    lax.fori_loop(0, NBANK * NROW, zinit, 0)
    for r in range(NACC):
        outv[pl.ds(r * L, L)] = zero

    laneconst = lax.iota(jnp.int32, L) - NLBL * L
    ones = jnp.full((L,), 1.0, jnp.float32)

    def issue(t):
        xv, mlv, rmv, sem = bufs[t % 2]
        off = base + t * TILE
        return (
            pltpu.make_async_copy(x_hbm.at[pl.ds(off, TILE)], xv, sem),
            pltpu.make_async_copy(ml_hbm.at[pl.ds(off, TILE)], mlv, sem),
            pltpu.make_async_copy(rm_hbm.at[pl.ds(off, TILE)], rmv, sem),
        )

    def start(handles):
        for h in handles:
            h.start()

    pending = issue(0)
    start(pending)
    for t in range(NT):
        if t + 1 < NT:
            nxt = issue(t + 1)
            start(nxt)
        else:
            nxt = None
        for h in pending:
            h.wait()
        xv, mlv, rmv, _ = bufs[t % 2]

        @functools.partial(plsc.parallel_loop, 0, TILE // L, unroll=8)
        def _(i):
            p = i * L
            xr = xv[pl.ds(p, L)]
            ml = mlv[pl.ds(p, L)]
            rm = rmv[pl.ds(p, L)]
            s = 1.0 / (1.0 + jnp.exp(-xr))
            bucket = (((rm << 3) + ml) << 4) + laneconst + ((i & (NBANK - 1)) << 9)
            plsc.addupdate_scatter(acc_s, [bucket], s)
            plsc.addupdate_scatter(acc_c, [bucket], ones)

        pending = nxt

    def fold(j, c):
        srow = zero
        crow = zero
        for k in range(NBANK):
            off = k * BANKSZ + j * L
            srow = srow + acc_s[pl.ds(off, L)]
            crow = crow + acc_c[pl.ds(off, L)]
        r = j >> 3
        fv = (j & (NLBL - 1)).astype(jnp.float32)
        o0 = r * L
        outv[pl.ds(o0, L)] = outv[pl.ds(o0, L)] + fv * srow
        o1 = (NREG + r) * L
        outv[pl.ds(o1, L)] = outv[pl.ds(o1, L)] + srow
        o2 = (2 * NREG + r) * L
        outv[pl.ds(o2, L)] = outv[pl.ds(o2, L)] + fv * crow
        return c

    lax.fori_loop(0, NROW, fold, 0)
    pltpu.sync_copy(outv, out_hbm.at[wid])


_sc_kernel = functools.partial(
    pl.kernel,
    out_type=jax.ShapeDtypeStruct((NW, NACC * L), jnp.float32),
    mesh=plsc.VectorSubcoreMesh(core_axis_name="c", subcore_axis_name="s",
                                num_cores=NC, num_subcores=NS),
    compiler_params=pltpu.CompilerParams(needs_layout_passes=False),
    scratch_types=[
        pltpu.VMEM((TILE,), jnp.float32),
        pltpu.VMEM((TILE,), jnp.int32),
        pltpu.VMEM((TILE,), jnp.int32),
        pltpu.VMEM((TILE,), jnp.float32),
        pltpu.VMEM((TILE,), jnp.int32),
        pltpu.VMEM((TILE,), jnp.int32),
        pltpu.VMEM((NBANK * BANKSZ,), jnp.float32),
        pltpu.VMEM((NBANK * BANKSZ,), jnp.float32),
        pltpu.VMEM((NACC * L,), jnp.float32),
        pltpu.SemaphoreType.DMA,
        pltpu.SemaphoreType.DMA,
    ],
)(_sc_body)


def _combine_body(p_ref, o_ref):
    total = 0.0
    for b in range(B):
        rows = p_ref[b * (NW // B):(b + 1) * (NW // B), :]
        mean_d = 0.0
        for r in range(NREG):
            i_sum = jnp.sum(rows[:, r * L:(r + 1) * L])
            a_sum = jnp.sum(rows[:, (NREG + r) * L:(NREG + r + 1) * L])
            y_sum = jnp.sum(rows[:, (2 * NREG + r) * L:(2 * NREG + r + 1) * L])
            mean_d += 2.0 * i_sum / (a_sum + y_sum)
        total += 1.0 - mean_d / NREG
    o_ref[...] = jnp.full((1, 1), total / B, jnp.float32)


def kernel(x, multi_label, region_map):
    xf = x.reshape(NVOX)
    mlf = multi_label.reshape(NVOX)
    rmf = region_map.reshape(NVOX)
    partials = _sc_kernel(xf, mlf, rmf)
    out = pl.pallas_call(
        _combine_body,
        out_shape=jax.ShapeDtypeStruct((1, 1), jnp.float32),
    )(partials)
    return out.reshape(())
